# baseline (speedup 1.0000x reference)
import functools

import numpy as np
import jax
import jax.numpy as jnp
from jax import lax
from jax.experimental import pallas as pl
from jax.experimental.pallas import tpu as pltpu

N, J, D, H, W = 4, 17, 64, 64, 64
V = D * H * W
NJ = N * J
NUM_SAMPLES = 16
BV = 2048
G = V // BV

_KEYS_U32 = np.array([
    [1832780943, 270669613],
    [64467757, 2916123636],
    [2465931498, 255383827],
    [3134548294, 894150801],
    [2954079971, 3276725750],
    [2765691542, 824333390],
    [2768684296, 3055579793],
    [2547012911, 1371500959],
    [1016697191, 2390192106],
    [1128875147, 2463678267],
    [1039196627, 1683848162],
    [246739928, 3519402408],
    [3114009986, 1419417030],
    [3514951389, 229662949],
    [2526883203, 3973959769],
    [991576401, 3935454969],
], dtype=np.uint64).astype(np.uint32)
_KEYS_I32 = np.ascontiguousarray(_KEYS_U32).view(np.int32).reshape(-1)

_ROTS = (13, 15, 26, 6, 17, 29, 16, 24, 13, 15, 26, 6, 17, 29, 16, 24,
         13, 15, 26, 6)
_TINY = np.float32(np.finfo(np.float32).tiny)
_IMAX = np.int32(2**31 - 1)


def _srl(x, r):
    return lax.shift_right_logical(x, jnp.full(x.shape, r, jnp.int32))


def _rotl(x, r):
    return (x << r) | _srl(x, 32 - r)


def _threefry_bits(c1, k1, k2):
    ks2 = k1 ^ k2 ^ np.int32(0x1BD11BDA)
    ks = (k1, k2, ks2)
    x0 = jnp.full(c1.shape, 0, jnp.int32) + k1
    x1 = c1 + k2
    for i in range(5):
        for r in _ROTS[4 * i:4 * i + 4]:
            x0 = x0 + x1
            x1 = _rotl(x1, r)
            x1 = x1 ^ x0
        x0 = x0 + ks[(i + 1) % 3]
        x1 = x1 + ks[(i + 2) % 3] + np.int32(i + 1)
    return x0 ^ x1


def _sample_kernel(keys_ref, logits_ref, samples_ref, lse_ref,
                   accv_ref, acci_ref, acce_ref):
    i = pl.program_id(0)
    s = pl.program_id(1)

    @pl.when(i == 0)
    def _init():
        accv_ref[s] = jnp.full((NJ, BV), -jnp.inf, jnp.float32)
        acci_ref[s] = jnp.zeros((NJ, BV), jnp.int32)

        @pl.when(s == 0)
        def _init_e():
            acce_ref[...] = jnp.zeros((NJ, BV), jnp.float32)

    logits_blk = logits_ref[...]
    row_iota = lax.broadcasted_iota(jnp.int32, (NJ, BV), 0)
    col_iota = lax.broadcasted_iota(jnp.int32, (NJ, BV), 1)
    vbase = i * BV
    c1 = row_iota * V + (col_iota + vbase)
    idx_mat = col_iota + vbase

    @pl.when(s == 0)
    def _accum_exp():
        acce_ref[...] = acce_ref[...] + jnp.exp(logits_blk)

    k1 = keys_ref[2 * s]
    k2 = keys_ref[2 * s + 1]
    bits = _threefry_bits(c1, k1, k2)
    fb = _srl(bits, 9) | np.int32(0x3F800000)
    f = lax.bitcast_convert_type(fb, jnp.float32) - np.float32(1.0)
    u = jnp.maximum(f, _TINY)
    t = logits_blk - jnp.log(-jnp.log(u))
    av = accv_ref[s]
    upd = t > av
    accv_ref[s] = jnp.where(upd, t, av)
    acci_ref[s] = jnp.where(upd, idx_mat, acci_ref[s])

    @pl.when(i == G - 1)
    def _finalize():
        av2 = accv_ref[s]
        m = jnp.max(av2, axis=1, keepdims=True)
        cand = jnp.where(av2 == m, acci_ref[s], _IMAX)
        samples_ref[...] = jnp.min(cand, axis=1, keepdims=True)[None]

        @pl.when(s == 0)
        def _fin_lse():
            tot = jnp.sum(acce_ref[...], axis=1, keepdims=True)
            lse_ref[...] = jnp.broadcast_to(jnp.log(tot), (NJ, NUM_SAMPLES))


def _run_sampler(logits2d):
    keys = jnp.asarray(_KEYS_I32)
    grid_spec = pltpu.PrefetchScalarGridSpec(
        num_scalar_prefetch=1,
        grid=(G, NUM_SAMPLES),
        in_specs=[pl.BlockSpec((NJ, BV), lambda i, s, keys: (0, i))],
        out_specs=[
            pl.BlockSpec((1, NJ, 1), lambda i, s, keys: (s, 0, 0)),
            pl.BlockSpec((NJ, NUM_SAMPLES), lambda i, s, keys: (0, 0)),
        ],
        scratch_shapes=[
            pltpu.VMEM((NUM_SAMPLES, NJ, BV), jnp.float32),
            pltpu.VMEM((NUM_SAMPLES, NJ, BV), jnp.int32),
            pltpu.VMEM((NJ, BV), jnp.float32),
        ],
    )
    samples_s, lse_b = pl.pallas_call(
        _sample_kernel,
        grid_spec=grid_spec,
        out_shape=[
            jax.ShapeDtypeStruct((NUM_SAMPLES, NJ, 1), jnp.int32),
            jax.ShapeDtypeStruct((NJ, NUM_SAMPLES), jnp.float32),
        ],
    )(keys, logits2d)
    samples_t = samples_s[:, :, 0].T
    return samples_t, lse_b


def kernel(preds, batch_joints, batch_joints_vis):
    logits2d = preds.reshape(NJ, V)
    samples_t, lse_b = _run_sampler(logits2d)

    logit_at = jnp.take_along_axis(logits2d, samples_t, axis=1)
    lp = logit_at - lse_b
    v = samples_t
    x = (v % W).astype(jnp.float32) / W - 0.5
    y = ((v // W) % H).astype(jnp.float32) / H - 0.5
    z = (v // (W * H)).astype(jnp.float32) / D - 0.5
    coords = jnp.stack((x, y, z), axis=-1)
    gt = batch_joints.reshape(NJ, 1, 3)
    vis = batch_joints_vis.reshape(NJ, 1, 3)
    d = (jnp.abs(coords - gt) * vis).sum(-1)
    l1 = d.reshape(N, J, NUM_SAMPLES).sum(axis=1)
    r = 1.0 / (-lp)
    rsum = r.reshape(N, J, NUM_SAMPLES).sum(axis=1)
    total = (l1 * rsum).sum() / (N * J * NUM_SAMPLES)
    return total


# speedup vs baseline: 1.0424x; 1.0424x over previous
import functools

import numpy as np
import jax
import jax.numpy as jnp
from jax import lax
from jax.experimental import pallas as pl
from jax.experimental.pallas import tpu as pltpu

N, J, D, H, W = 4, 17, 64, 64, 64
V = D * H * W
NJ = N * J
NUM_SAMPLES = 16
BV = 4096
G = V // BV

_KEYS_U32 = np.array([
    [1832780943, 270669613],
    [64467757, 2916123636],
    [2465931498, 255383827],
    [3134548294, 894150801],
    [2954079971, 3276725750],
    [2765691542, 824333390],
    [2768684296, 3055579793],
    [2547012911, 1371500959],
    [1016697191, 2390192106],
    [1128875147, 2463678267],
    [1039196627, 1683848162],
    [246739928, 3519402408],
    [3114009986, 1419417030],
    [3514951389, 229662949],
    [2526883203, 3973959769],
    [991576401, 3935454969],
], dtype=np.uint64).astype(np.uint32)
_KEYS_I32 = np.ascontiguousarray(_KEYS_U32).view(np.int32).reshape(-1)

_ROTS = (13, 15, 26, 6, 17, 29, 16, 24, 13, 15, 26, 6, 17, 29, 16, 24,
         13, 15, 26, 6)
_TINY = np.float32(np.finfo(np.float32).tiny)
_IMAX = np.int32(2**31 - 1)


def _srl(x, r):
    return lax.shift_right_logical(x, jnp.full(x.shape, r, jnp.int32))


def _rotl(x, r):
    return (x << r) | _srl(x, 32 - r)


def _threefry_bits(c1, k1, k2):
    ks2 = k1 ^ k2 ^ np.int32(0x1BD11BDA)
    ks = (k1, k2, ks2)
    x0 = jnp.full(c1.shape, 0, jnp.int32) + k1
    x1 = c1 + k2
    for i in range(5):
        for r in _ROTS[4 * i:4 * i + 4]:
            x0 = x0 + x1
            x1 = _rotl(x1, r)
            x1 = x1 ^ x0
        x0 = x0 + ks[(i + 1) % 3]
        x1 = x1 + ks[(i + 2) % 3] + np.int32(i + 1)
    return x0 ^ x1


def _sample_kernel(keys_ref, logits_ref, samples_ref, lse_ref,
                   accv_ref, acci_ref, acce_ref):
    i = pl.program_id(0)

    @pl.when(i == 0)
    def _init():
        accv_ref[...] = jnp.full((NUM_SAMPLES, NJ, BV), -jnp.inf, jnp.float32)
        acci_ref[...] = jnp.zeros((NUM_SAMPLES, NJ, BV), jnp.int32)
        acce_ref[...] = jnp.zeros((NJ, BV), jnp.float32)

    logits_blk = logits_ref[...]
    row_iota = lax.broadcasted_iota(jnp.int32, (NJ, BV), 0)
    col_iota = lax.broadcasted_iota(jnp.int32, (NJ, BV), 1)
    vbase = i * BV
    c1 = row_iota * V + (col_iota + vbase)
    idx_mat = col_iota + vbase

    acce_ref[...] = acce_ref[...] + jnp.exp(logits_blk)

    def body(s, _):
        k1 = keys_ref[2 * s]
        k2 = keys_ref[2 * s + 1]
        bits = _threefry_bits(c1, k1, k2)
        fb = _srl(bits, 9) | np.int32(0x3F800000)
        f = lax.bitcast_convert_type(fb, jnp.float32) - np.float32(1.0)
        u = jnp.maximum(f, _TINY)
        t = logits_blk - jnp.log(-jnp.log(u))
        av = accv_ref[s]
        upd = t > av
        accv_ref[s] = jnp.where(upd, t, av)
        acci_ref[s] = jnp.where(upd, idx_mat, acci_ref[s])
        return 0

    lax.fori_loop(0, NUM_SAMPLES, body, 0, unroll=False)

    @pl.when(i == G - 1)
    def _finalize():
        for s in range(NUM_SAMPLES):
            av = accv_ref[s]
            m = jnp.max(av, axis=1, keepdims=True)
            cand = jnp.where(av == m, acci_ref[s], _IMAX)
            samples_ref[:, s:s + 1] = jnp.min(cand, axis=1, keepdims=True)
        tot = jnp.sum(acce_ref[...], axis=1, keepdims=True)
        lse_ref[...] = jnp.broadcast_to(jnp.log(tot), (NJ, NUM_SAMPLES))


def _run_sampler(logits2d):
    keys = jnp.asarray(_KEYS_I32)
    grid_spec = pltpu.PrefetchScalarGridSpec(
        num_scalar_prefetch=1,
        grid=(G,),
        in_specs=[pl.BlockSpec((NJ, BV), lambda i, keys: (0, i))],
        out_specs=[
            pl.BlockSpec((NJ, NUM_SAMPLES), lambda i, keys: (0, 0)),
            pl.BlockSpec((NJ, NUM_SAMPLES), lambda i, keys: (0, 0)),
        ],
        scratch_shapes=[
            pltpu.VMEM((NUM_SAMPLES, NJ, BV), jnp.float32),
            pltpu.VMEM((NUM_SAMPLES, NJ, BV), jnp.int32),
            pltpu.VMEM((NJ, BV), jnp.float32),
        ],
    )
    samples_t, lse_b = pl.pallas_call(
        _sample_kernel,
        grid_spec=grid_spec,
        out_shape=[
            jax.ShapeDtypeStruct((NJ, NUM_SAMPLES), jnp.int32),
            jax.ShapeDtypeStruct((NJ, NUM_SAMPLES), jnp.float32),
        ],
    )(keys, logits2d)
    return samples_t, lse_b


def kernel(preds, batch_joints, batch_joints_vis):
    logits2d = preds.reshape(NJ, V)
    samples_t, lse_b = _run_sampler(logits2d)

    logit_at = jnp.take_along_axis(logits2d, samples_t, axis=1)
    lp = logit_at - lse_b
    v = samples_t
    x = (v % W).astype(jnp.float32) / W - 0.5
    y = ((v // W) % H).astype(jnp.float32) / H - 0.5
    z = (v // (W * H)).astype(jnp.float32) / D - 0.5
    coords = jnp.stack((x, y, z), axis=-1)
    gt = batch_joints.reshape(NJ, 1, 3)
    vis = batch_joints_vis.reshape(NJ, 1, 3)
    d = (jnp.abs(coords - gt) * vis).sum(-1)
    l1 = d.reshape(N, J, NUM_SAMPLES).sum(axis=1)
    r = 1.0 / (-lp)
    rsum = r.reshape(N, J, NUM_SAMPLES).sum(axis=1)
    total = (l1 * rsum).sum() / (N * J * NUM_SAMPLES)
    return total
